# baseline (speedup 1.0000x reference)
import functools

import jax
import jax.numpy as jnp
from jax import lax
from jax.experimental import pallas as pl
from jax.experimental.pallas import tpu as pltpu
from jax.experimental.pallas import tpu_sc as plsc

_NBINS = 15
_C = 100
_CHUNK = 160
_GROUPS = _CHUNK // 16
_LANES = 16
_NW = 32


def _sc_body(nchunks, logits_ref, labels_ref,
             cnt_out, cnf_out, acc_out, nin_out, ncor_out,
             bufa, bufb, labbuf, ebuf, cntp, cnfp, accp, ninp, ncorp,
             cnt, cnf, acc, nin, ncor, sem0, sem1):
    wid = lax.axis_index("s") * 2 + lax.axis_index("c")
    iota = lax.iota(jnp.int32, _LANES)
    ones = jnp.ones((_LANES,), jnp.float32)
    zeros = jnp.zeros((_LANES,), jnp.float32)
    lane0 = iota == 0
    mall = iota < _LANES

    def _zero_priv(r, carry):
        sl = pl.ds(r * _LANES, _LANES)
        cntp[sl] = zeros
        cnfp[sl] = zeros
        accp[sl] = zeros
        return carry
    lax.fori_loop(0, _NBINS * _C, _zero_priv, 0)

    def _zero_small(r, carry):
        sl = pl.ds(r * _LANES, _LANES)
        ninp[sl] = zeros
        ncorp[sl] = zeros
        return carry
    lax.fori_loop(0, _C, _zero_small, 0)

    def _scat_add(ref, idx, x, mask):
        plsc.addupdate_scatter(ref, [idx], x, mask=mask)

    def _do_group(buf, g, carry):
        row_vec = iota + g * _LANES

        def _pass_a(c, rc):
            rs, rm = rc
            cvec = jnp.full((_LANES,), c)
            v = plsc.load_gather(buf, [row_vec, cvec])
            ee = jnp.exp(v)
            ebuf[pl.ds(c * _LANES, _LANES)] = ee
            return (rs + ee, jnp.maximum(rm, ee))

        rsum, rmax = lax.fori_loop(0, _C, _pass_a, (zeros, zeros),
                                   unroll=4)
        rcps = ones / rsum
        pmax = rmax * rcps

        def _pass_b(c, bc):
            choice, base = bc
            ee = ebuf[pl.ds(c * _LANES, _LANES)]
            p = ee * rcps
            u = jnp.minimum((p * 15.0).astype(jnp.int32), 14)
            idx = u * (_C * _LANES) + base
            _scat_add(cntp, idx, ones, mall)
            _scat_add(cnfp, idx, p, mall)
            cvec = jnp.full((_LANES,), c)
            choice = jnp.minimum(choice, jnp.where(p == pmax, cvec, 1000))
            return (choice, base + _LANES)

        choice, _ = lax.fori_loop(
            0, _C, _pass_b, (jnp.full((_LANES,), 1000, jnp.int32), iota),
            unroll=4)

        lv = labbuf[pl.ds(g * _LANES, _LANES)]
        e_lab = plsc.load_gather(ebuf, [lv * _LANES + iota])
        p_lab = e_lab * rcps
        ulab = jnp.minimum((p_lab * 15.0).astype(jnp.int32), 14)
        _scat_add(accp, ulab * (_C * _LANES) + lv * _LANES + iota,
                  ones, mall)
        _scat_add(ninp, lv * _LANES + iota, ones, mall)
        eq_vec = jnp.where(choice == lv, 1.0, 0.0).astype(jnp.float32)
        _scat_add(ncorp, lv * _LANES + iota, eq_vec, mall)
        return carry

    bufs = (bufa, bufb)
    sems = (sem0, sem1)

    def _start(c, b):
        pltpu.make_async_copy(logits_ref.at[c], bufs[b], sems[b]).start()

    def _wait(c, b):
        pltpu.make_async_copy(logits_ref.at[c], bufs[b], sems[b]).wait()

    _start(wid, 0)

    def _outer(i, carry):
        for b in range(2):
            j = 2 * i + b
            c = wid + _NW * j

            @pl.when(c < nchunks)
            def _(c=c, b=b):
                cnext = c + _NW

                @pl.when(cnext < nchunks)
                def _():
                    _start(cnext, 1 - b)

                _wait(c, b)
                pltpu.sync_copy(labels_ref.at[c], labbuf)
                lax.fori_loop(0, _GROUPS,
                              functools.partial(_do_group, bufs[b]), 0)
        return carry

    njw = (nchunks + _NW - 1) // _NW
    lax.fori_loop(0, (njw + 1) // 2, _outer, 0)

    def _zero_merged(r, carry):
        sl = pl.ds(r * _LANES, _LANES)
        cnt[sl] = zeros
        cnf[sl] = zeros
        acc[sl] = zeros
        return carry
    lax.fori_loop(0, _NBINS * 8, _zero_merged, 0)
    for lq in range(8):
        sl = pl.ds(lq * _LANES, _LANES)
        nin[sl] = zeros
        ncor[sl] = zeros

    def _lanesum(x):
        for k in (8, 4, 2, 1):
            x = x + x.at[iota ^ k].get(mode="promise_in_bounds")
        return x

    def _merge_u(u, carry):
        def _merge_c(cm, carry2):
            dst = jnp.full((_LANES,), u * 128 + cm)
            sl = pl.ds((u * _C + cm) * _LANES, _LANES)
            plsc.store_scatter(cnt, [dst], _lanesum(cntp[sl]), mask=lane0)
            plsc.store_scatter(cnf, [dst], _lanesum(cnfp[sl]), mask=lane0)
            plsc.store_scatter(acc, [dst], _lanesum(accp[sl]), mask=lane0)
            return carry2
        return lax.fori_loop(0, _C, _merge_c, carry)
    lax.fori_loop(0, _NBINS, _merge_u, 0)

    def _merge_small(i, carry):
        dst = jnp.full((_LANES,), i)
        sl = pl.ds(i * _LANES, _LANES)
        plsc.store_scatter(nin, [dst], _lanesum(ninp[sl]), mask=lane0)
        plsc.store_scatter(ncor, [dst], _lanesum(ncorp[sl]), mask=lane0)
        return carry
    lax.fori_loop(0, _C, _merge_small, 0)

    pltpu.sync_copy(cnt, cnt_out.at[wid])
    pltpu.sync_copy(cnf, cnf_out.at[wid])
    pltpu.sync_copy(acc, acc_out.at[wid])
    pltpu.sync_copy(nin, nin_out.at[wid])
    pltpu.sync_copy(ncor, ncor_out.at[wid])


def _tc_final_body(nrows, cnt_ref, cnf_ref, acc_ref, nin_ref, ncor_ref,
                   sce_ref, cacc_ref):
    cnt = jnp.sum(cnt_ref[...], axis=0)
    cnf = jnp.sum(cnf_ref[...], axis=0)
    acc = jnp.sum(acc_ref[...], axis=0)
    nin = jnp.sum(nin_ref[...], axis=0, keepdims=True)
    ncor = jnp.sum(ncor_ref[...], axis=0, keepdims=True)
    prop = cnt / float(nrows)
    safe = jnp.maximum(cnt, 1.0)
    contrib = jnp.where(cnt > 0.0,
                        jnp.abs(cnf / safe - acc / safe) * prop, 0.0)
    sce_ref[...] = jnp.sum(contrib, axis=0, keepdims=True)
    cacc_ref[...] = ncor / nin


def kernel(logits, labels):
    N, C = logits.shape
    nchunks = N // _CHUNK
    logits3 = logits.reshape(nchunks, _CHUNK, C)
    labels2 = labels.reshape(nchunks, _CHUNK)

    mesh = plsc.VectorSubcoreMesh(core_axis_name="c", subcore_axis_name="s")
    sck = functools.partial(
        pl.kernel,
        mesh=mesh,
        compiler_params=pltpu.CompilerParams(needs_layout_passes=False),
        out_type=[
            jax.ShapeDtypeStruct((_NW, _NBINS * 128), jnp.float32),
            jax.ShapeDtypeStruct((_NW, _NBINS * 128), jnp.float32),
            jax.ShapeDtypeStruct((_NW, _NBINS * 128), jnp.float32),
            jax.ShapeDtypeStruct((_NW, 128), jnp.float32),
            jax.ShapeDtypeStruct((_NW, 128), jnp.float32),
        ],
        scratch_types=[
            pltpu.VMEM((_CHUNK, _C), jnp.float32),
            pltpu.VMEM((_CHUNK, _C), jnp.float32),
            pltpu.VMEM((_CHUNK,), jnp.int32),
            pltpu.VMEM((_C * _LANES,), jnp.float32),
            pltpu.VMEM((_NBINS * _C * _LANES,), jnp.float32),
            pltpu.VMEM((_NBINS * _C * _LANES,), jnp.float32),
            pltpu.VMEM((_NBINS * _C * _LANES,), jnp.float32),
            pltpu.VMEM((_C * _LANES,), jnp.float32),
            pltpu.VMEM((_C * _LANES,), jnp.float32),
            pltpu.VMEM((_NBINS * 128,), jnp.float32),
            pltpu.VMEM((_NBINS * 128,), jnp.float32),
            pltpu.VMEM((_NBINS * 128,), jnp.float32),
            pltpu.VMEM((128,), jnp.float32),
            pltpu.VMEM((128,), jnp.float32),
            pltpu.SemaphoreType.DMA,
            pltpu.SemaphoreType.DMA,
        ],
    )(functools.partial(_sc_body, nchunks))
    cnt, cnf, acc, nin, ncor = sck(logits3, labels2)
    cnt = cnt.reshape(_NW, _NBINS, 128)
    cnf = cnf.reshape(_NW, _NBINS, 128)
    acc = acc.reshape(_NW, _NBINS, 128)

    out = pl.pallas_call(
        functools.partial(_tc_final_body, N),
        out_shape=[
            jax.ShapeDtypeStruct((1, 128), jnp.float32),
            jax.ShapeDtypeStruct((1, 128), jnp.float32),
        ],
    )(cnt, cnf, acc, nin, ncor)
    return (out[0][0, :C], out[1][0, :C])


# speedup vs baseline: 2.6344x; 2.6344x over previous
import functools

import jax
import jax.numpy as jnp
from jax import lax
from jax.experimental import pallas as pl
from jax.experimental.pallas import tpu as pltpu
from jax.experimental.pallas import tpu_sc as plsc

_NBINS = 15
_C = 100
_CHUNK = 160
_LANES = 16
_NW = 32
_NT = 64000
_RT = 2000


def _sc_body(nchunks, logits_ref, labels_ref,
             cnt_out, cnf_out, acc_out, nin_out, ncor_out,
             buf, labbuf, cnt, cnf, acc, nin, ncor):
    wid = lax.axis_index("s") * 2 + lax.axis_index("c")
    iota = lax.iota(jnp.int32, _LANES)
    ones = jnp.ones((_LANES,), jnp.float32)
    zeros = jnp.zeros((_LANES,), jnp.float32)
    lane0 = iota == 0
    mall = iota < _LANES
    m6 = iota >= 12

    def _zero_row(r, carry):
        sl = pl.ds(r * _LANES, _LANES)
        cnt[sl] = zeros
        cnf[sl] = zeros
        acc[sl] = zeros
        return carry
    lax.fori_loop(0, _NBINS * 8, _zero_row, 0)
    for lq in range(8):
        sl = pl.ds(lq * _LANES, _LANES)
        nin[sl] = zeros
        ncor[sl] = zeros

    def _scat_add(ref, idx, x, mask):
        plsc.addupdate_scatter(ref, [idx], x, mask=mask)

    def _shuf(x, k):
        return x.at[iota ^ k].get(mode="promise_in_bounds")

    def _allreduce(x, op):
        for k in (8, 4, 2, 1):
            x = op(x, _shuf(x, k))
        return x

    def _do_row(r, carry):
        v = [buf[r, pl.ds(16 * j, _LANES)] for j in range(6)]
        v6 = buf[r, pl.ds(84, _LANES)]
        mm = v[0]
        for j in range(1, 6):
            mm = jnp.maximum(mm, v[j])
        mm = jnp.maximum(mm, v6)
        mvec = _allreduce(mm, jnp.maximum)
        e = [jnp.exp(vj - mvec) for vj in v]
        e6 = jnp.exp(v6 - mvec)
        ssum = e[0]
        for j in range(1, 6):
            ssum = ssum + e[j]
        ssum = ssum + jnp.where(m6, e6, zeros)
        svec = _allreduce(ssum, jnp.add)
        p = [ej / svec for ej in e]
        p6 = e6 / svec
        pmax = ones / svec
        tmin = jnp.full((_LANES,), 1000, jnp.int32)
        for j in range(6):
            tmin = jnp.minimum(tmin, jnp.where(p[j] == pmax, iota + 16 * j, 1000))
        tmin = jnp.minimum(tmin, jnp.where((p6 == pmax) & m6, iota + 84, 1000))
        choice = _allreduce(tmin, jnp.minimum)

        for j in range(6):
            u = jnp.minimum((p[j] * 15.0).astype(jnp.int32), 14)
            idx = (u << 7) + (iota + 16 * j)
            _scat_add(cnt, idx, ones, mall)
            _scat_add(cnf, idx, p[j], mall)
        u6 = jnp.minimum((p6 * 15.0).astype(jnp.int32), 14)
        idx6 = (u6 << 7) + (iota + 84)
        _scat_add(cnt, idx6, ones, m6)
        _scat_add(cnf, idx6, p6, m6)

        lv = labbuf[pl.ds(r, _LANES)]
        lab_vec = lv.at[jnp.zeros((_LANES,), jnp.int32)].get(
            mode="promise_in_bounds")
        vlab = plsc.load_gather(buf, [jnp.full((_LANES,), r), lab_vec])
        conf_lab = jnp.exp(vlab - mvec) / svec
        ulab = jnp.minimum((conf_lab * 15.0).astype(jnp.int32), 14)
        _scat_add(acc, (ulab << 7) + lab_vec, ones, lane0)
        _scat_add(nin, lab_vec, ones, lane0)
        eq_vec = jnp.where(choice == lab_vec, 1.0, 0.0).astype(jnp.float32)
        _scat_add(ncor, lab_vec, eq_vec, lane0)
        return carry

    def _chunk_iter(j, carry):
        c = wid + _NW * j

        @pl.when(c < nchunks)
        def _():
            pltpu.sync_copy(logits_ref.at[c], buf)
            pltpu.sync_copy(labels_ref.at[c], labbuf)
            lax.fori_loop(0, _CHUNK, _do_row, 0)
        return carry
    njw = (nchunks + _NW - 1) // _NW
    lax.fori_loop(0, njw, _chunk_iter, 0)

    pltpu.sync_copy(cnt, cnt_out.at[wid])
    pltpu.sync_copy(cnf, cnf_out.at[wid])
    pltpu.sync_copy(acc, acc_out.at[wid])
    pltpu.sync_copy(nin, nin_out.at[wid])
    pltpu.sync_copy(ncor, ncor_out.at[wid])


def _tc_hist_body(nblocks, logits_ref, labels_ref, bs_ref, bv_ref,
                  g_out, s_out, a_out):
    i = pl.program_id(0)
    R, C = logits_ref.shape
    x = logits_ref[...]
    m = jnp.max(x, axis=1, keepdims=True)
    e = jnp.exp(x - m)
    p = e / jnp.sum(e, axis=1, keepdims=True)

    lab = labels_ref[...]
    iota = lax.broadcasted_iota(jnp.int32, (R, C), 1)
    lab_oh = (iota == lab).astype(jnp.float32)

    pm = jnp.max(p, axis=1, keepdims=True)
    choice = jnp.min(jnp.where(p == pm, iota, C), axis=1, keepdims=True)
    eqf = (choice == lab).astype(jnp.float32)

    conf_lab = jnp.sum(p * lab_oh, axis=1, keepdims=True)
    bv = bv_ref[...]
    ecat = jnp.concatenate(
        [(conf_lab > bv).astype(jnp.float32), eqf, jnp.ones_like(eqf)],
        axis=1)
    a_blk = lax.dot_general(ecat, lab_oh, (((0,), (0,)), ((), ())),
                            preferred_element_type=jnp.float32)

    p3 = p.reshape(R // 8, 8, C)
    glist, slist = [], []
    for k in range(_NBINS):
        t = bs_ref[0, k]
        mk = p3 > t
        glist.append(jnp.sum(mk.astype(jnp.float32), axis=0))
        slist.append(jnp.sum(jnp.where(mk, p3, 0.0), axis=0))
    g_blk = jnp.concatenate(glist, axis=0)
    s_blk = jnp.concatenate(slist, axis=0)

    @pl.when(i == 0)
    def _init():
        g_out[...] = jnp.zeros_like(g_out)
        s_out[...] = jnp.zeros_like(s_out)
        a_out[...] = jnp.zeros_like(a_out)

    g_out[...] += g_blk
    s_out[...] += s_blk
    a_out[...] += a_blk


def _fin_body(nrows, g_ref, s_ref, a_ref,
              cnts_ref, cnfs_ref, accs_ref, nins_ref, ncors_ref,
              sce_ref, cacc_ref):
    gm = jnp.sum(g_ref[...].reshape(_NBINS, 8, _C), axis=1)
    sm = jnp.sum(s_ref[...].reshape(_NBINS, 8, _C), axis=1)
    am = a_ref[...]
    zrow = jnp.zeros((1, _C), jnp.float32)
    cnt_sc = jnp.sum(cnts_ref[...], axis=0)[:, :_C]
    cnf_sc = jnp.sum(cnfs_ref[...], axis=0)[:, :_C]
    acc_sc = jnp.sum(accs_ref[...], axis=0)[:, :_C]
    nin_sc = jnp.sum(nins_ref[...], axis=0, keepdims=True)[:, :_C]
    ncor_sc = jnp.sum(ncors_ref[...], axis=0, keepdims=True)[:, :_C]

    counts = gm - jnp.concatenate([gm[1:, :], zrow], axis=0) + cnt_sc
    confs = sm - jnp.concatenate([sm[1:, :], zrow], axis=0) + cnf_sc
    accn = am[0:_NBINS, :] - am[1:_NBINS + 1, :] + acc_sc
    ncor = am[_NBINS + 1:_NBINS + 2, :] + ncor_sc
    nin = am[_NBINS + 2:_NBINS + 3, :] + nin_sc

    prop = counts / float(nrows)
    safe = jnp.maximum(counts, 1.0)
    contrib = jnp.where(counts > 0.0,
                        jnp.abs(confs / safe - accn / safe) * prop, 0.0)
    sce_ref[...] = jnp.sum(contrib, axis=0, keepdims=True)
    cacc_ref[...] = ncor / nin


def kernel(logits, labels):
    N, C = logits.shape
    ns = N - _NT
    nchunks = ns // _CHUNK
    logits_sc = logits[_NT:].reshape(nchunks, _CHUNK, C)
    labels_sc = jnp.pad(labels[_NT:].reshape(nchunks, _CHUNK),
                        ((0, 0), (0, 22)))

    mesh = plsc.VectorSubcoreMesh(core_axis_name="c", subcore_axis_name="s")
    sck = functools.partial(
        pl.kernel,
        mesh=mesh,
        compiler_params=pltpu.CompilerParams(needs_layout_passes=False),
        out_type=[
            jax.ShapeDtypeStruct((_NW, _NBINS * 128), jnp.float32),
            jax.ShapeDtypeStruct((_NW, _NBINS * 128), jnp.float32),
            jax.ShapeDtypeStruct((_NW, _NBINS * 128), jnp.float32),
            jax.ShapeDtypeStruct((_NW, 128), jnp.float32),
            jax.ShapeDtypeStruct((_NW, 128), jnp.float32),
        ],
        scratch_types=[
            pltpu.VMEM((_CHUNK, _C), jnp.float32),
            pltpu.VMEM((_CHUNK + 22,), jnp.int32),
            pltpu.VMEM((_NBINS * 128,), jnp.float32),
            pltpu.VMEM((_NBINS * 128,), jnp.float32),
            pltpu.VMEM((_NBINS * 128,), jnp.float32),
            pltpu.VMEM((128,), jnp.float32),
            pltpu.VMEM((128,), jnp.float32),
        ],
    )(functools.partial(_sc_body, nchunks))
    cnt, cnf, acc, nin, ncor = sck(logits_sc, labels_sc)
    cnt = cnt.reshape(_NW, _NBINS, 128)
    cnf = cnf.reshape(_NW, _NBINS, 128)
    acc = acc.reshape(_NW, _NBINS, 128)

    nblocks = _NT // _RT
    bounds = jnp.linspace(0.0, 1.0, _NBINS + 1).reshape(1, _NBINS + 1)
    labels_tc = labels[:_NT].reshape(_NT, 1)

    g, s, a = pl.pallas_call(
        functools.partial(_tc_hist_body, nblocks),
        grid=(nblocks,),
        in_specs=[
            pl.BlockSpec((_RT, C), lambda i: (i, 0)),
            pl.BlockSpec((_RT, 1), lambda i: (i, 0)),
            pl.BlockSpec(memory_space=pltpu.SMEM),
            pl.BlockSpec((1, _NBINS + 1), lambda i: (0, 0)),
        ],
        out_specs=[
            pl.BlockSpec((_NBINS * 8, C), lambda i: (0, 0)),
            pl.BlockSpec((_NBINS * 8, C), lambda i: (0, 0)),
            pl.BlockSpec((_NBINS + 3, C), lambda i: (0, 0)),
        ],
        out_shape=[
            jax.ShapeDtypeStruct((_NBINS * 8, C), jnp.float32),
            jax.ShapeDtypeStruct((_NBINS * 8, C), jnp.float32),
            jax.ShapeDtypeStruct((_NBINS + 3, C), jnp.float32),
        ],
    )(logits[:_NT], labels_tc, bounds, bounds)

    out = pl.pallas_call(
        functools.partial(_fin_body, N),
        out_shape=[
            jax.ShapeDtypeStruct((1, C), jnp.float32),
            jax.ShapeDtypeStruct((1, C), jnp.float32),
        ],
    )(g, s, a, cnt, cnf, acc, nin, ncor)
    return (out[0].reshape(C), out[1].reshape(C))


# speedup vs baseline: 3.1977x; 1.2138x over previous
import functools

import jax
import jax.numpy as jnp
from jax import lax
from jax.experimental import pallas as pl
from jax.experimental.pallas import tpu as pltpu
from jax.experimental.pallas import tpu_sc as plsc

_NBINS = 15
_C = 100
_CHUNK = 160
_LANES = 16
_NW = 32
_NT = 64000
_RT = 2000


def _sc_body(c0, nchunks, logits_ref, labels_ref,
             cnt_out, cnf_out, acc_out, nin_out, ncor_out,
             buf, labbuf, cnt, cnf, acc, nin, ncor):
    wid = lax.axis_index("s") * 2 + lax.axis_index("c")
    iota = lax.iota(jnp.int32, _LANES)
    ones = jnp.ones((_LANES,), jnp.float32)
    zeros = jnp.zeros((_LANES,), jnp.float32)
    lane0 = iota == 0
    mall = iota < _LANES
    m6 = iota >= 12

    def _zero_row(r, carry):
        sl = pl.ds(r * _LANES, _LANES)
        cnt[sl] = zeros
        cnf[sl] = zeros
        acc[sl] = zeros
        return carry
    lax.fori_loop(0, _NBINS * 8, _zero_row, 0)
    for lq in range(8):
        sl = pl.ds(lq * _LANES, _LANES)
        nin[sl] = zeros
        ncor[sl] = zeros

    def _scat_add(ref, idx, x, mask):
        plsc.addupdate_scatter(ref, [idx], x, mask=mask)

    def _shuf(x, k):
        return x.at[iota ^ k].get(mode="promise_in_bounds")

    def _allreduce(x, op):
        for k in (8, 4, 2, 1):
            x = op(x, _shuf(x, k))
        return x

    def _do_row(r, carry):
        v = [buf[r, pl.ds(16 * j, _LANES)] for j in range(6)]
        v6 = buf[r, pl.ds(84, _LANES)]
        mm = v[0]
        for j in range(1, 6):
            mm = jnp.maximum(mm, v[j])
        mm = jnp.maximum(mm, v6)
        mvec = _allreduce(mm, jnp.maximum)
        e = [jnp.exp(vj - mvec) for vj in v]
        e6 = jnp.exp(v6 - mvec)
        ssum = e[0]
        for j in range(1, 6):
            ssum = ssum + e[j]
        ssum = ssum + jnp.where(m6, e6, zeros)
        svec = _allreduce(ssum, jnp.add)
        p = [ej / svec for ej in e]
        p6 = e6 / svec
        pmax = ones / svec
        tmin = jnp.full((_LANES,), 1000, jnp.int32)
        for j in range(6):
            tmin = jnp.minimum(tmin, jnp.where(p[j] == pmax, iota + 16 * j, 1000))
        tmin = jnp.minimum(tmin, jnp.where((p6 == pmax) & m6, iota + 84, 1000))
        choice = _allreduce(tmin, jnp.minimum)

        for j in range(6):
            u = jnp.minimum((p[j] * 15.0).astype(jnp.int32), 14)
            idx = (u << 7) + (iota + 16 * j)
            _scat_add(cnt, idx, ones, mall)
            _scat_add(cnf, idx, p[j], mall)
        u6 = jnp.minimum((p6 * 15.0).astype(jnp.int32), 14)
        idx6 = (u6 << 7) + (iota + 84)
        _scat_add(cnt, idx6, ones, m6)
        _scat_add(cnf, idx6, p6, m6)

        rb = jnp.minimum(r, _CHUNK - _LANES)
        lv = labbuf[pl.ds(rb, _LANES)]
        lab_vec = lv.at[jnp.full((_LANES,), r - rb)].get(
            mode="promise_in_bounds")
        vlab = plsc.load_gather(buf, [jnp.full((_LANES,), r), lab_vec])
        conf_lab = jnp.exp(vlab - mvec) / svec
        ulab = jnp.minimum((conf_lab * 15.0).astype(jnp.int32), 14)
        _scat_add(acc, (ulab << 7) + lab_vec, ones, lane0)
        _scat_add(nin, lab_vec, ones, lane0)
        eq_vec = jnp.where(choice == lab_vec, 1.0, 0.0).astype(jnp.float32)
        _scat_add(ncor, lab_vec, eq_vec, lane0)
        return carry

    def _chunk_iter(j, carry):
        c = c0 + wid + _NW * j

        @pl.when(c < nchunks)
        def _():
            pltpu.sync_copy(logits_ref.at[c], buf)
            pltpu.sync_copy(labels_ref.at[c], labbuf)
            lax.fori_loop(0, _CHUNK, _do_row, 0)
        return carry
    njw = (nchunks - c0 + _NW - 1) // _NW
    lax.fori_loop(0, njw, _chunk_iter, 0)

    pltpu.sync_copy(cnt, cnt_out.at[wid])
    pltpu.sync_copy(cnf, cnf_out.at[wid])
    pltpu.sync_copy(acc, acc_out.at[wid])
    pltpu.sync_copy(nin, nin_out.at[wid])
    pltpu.sync_copy(ncor, ncor_out.at[wid])


def _tc_hist_body(nblocks, logits_ref, labels_ref, bs_ref, bv_ref,
                  g_out, s_out, a_out):
    i = pl.program_id(0)
    R, C = logits_ref.shape
    x = logits_ref[...]
    m = jnp.max(x, axis=1, keepdims=True)
    e = jnp.exp(x - m)
    p = e / jnp.sum(e, axis=1, keepdims=True)

    lab = labels_ref[...]
    iota = lax.broadcasted_iota(jnp.int32, (R, C), 1)
    lab_oh = (iota == lab).astype(jnp.float32)

    pm = jnp.max(p, axis=1, keepdims=True)
    choice = jnp.min(jnp.where(p == pm, iota, C), axis=1, keepdims=True)
    eqf = (choice == lab).astype(jnp.float32)

    conf_lab = jnp.sum(p * lab_oh, axis=1, keepdims=True)
    bv = bv_ref[...]
    ecat = jnp.concatenate(
        [(conf_lab > bv).astype(jnp.float32), eqf, jnp.ones_like(eqf)],
        axis=1)
    a_blk = lax.dot_general(ecat, lab_oh, (((0,), (0,)), ((), ())),
                            preferred_element_type=jnp.float32)

    p3 = p.reshape(R // 8, 8, C)
    glist, slist = [], []
    for k in range(_NBINS):
        t = bs_ref[0, k]
        mk = p3 > t
        glist.append(jnp.sum(mk.astype(jnp.float32), axis=0))
        slist.append(jnp.sum(jnp.where(mk, p3, 0.0), axis=0))
    g_blk = jnp.concatenate(glist, axis=0)
    s_blk = jnp.concatenate(slist, axis=0)

    @pl.when(i == 0)
    def _init():
        g_out[...] = jnp.zeros_like(g_out)
        s_out[...] = jnp.zeros_like(s_out)
        a_out[...] = jnp.zeros_like(a_out)

    g_out[...] += g_blk
    s_out[...] += s_blk
    a_out[...] += a_blk


def _fin_body(nrows, g_ref, s_ref, a_ref,
              cnts_ref, cnfs_ref, accs_ref, nins_ref, ncors_ref,
              sce_ref, cacc_ref):
    gm = jnp.sum(g_ref[...].reshape(_NBINS, 8, _C), axis=1)
    sm = jnp.sum(s_ref[...].reshape(_NBINS, 8, _C), axis=1)
    am = a_ref[...]
    zrow = jnp.zeros((1, _C), jnp.float32)
    cnt_sc = jnp.sum(cnts_ref[...], axis=0)[:, :_C]
    cnf_sc = jnp.sum(cnfs_ref[...], axis=0)[:, :_C]
    acc_sc = jnp.sum(accs_ref[...], axis=0)[:, :_C]
    nin_sc = jnp.sum(nins_ref[...], axis=0, keepdims=True)[:, :_C]
    ncor_sc = jnp.sum(ncors_ref[...], axis=0, keepdims=True)[:, :_C]

    counts = gm - jnp.concatenate([gm[1:, :], zrow], axis=0) + cnt_sc
    confs = sm - jnp.concatenate([sm[1:, :], zrow], axis=0) + cnf_sc
    accn = am[0:_NBINS, :] - am[1:_NBINS + 1, :] + acc_sc
    ncor = am[_NBINS + 1:_NBINS + 2, :] + ncor_sc
    nin = am[_NBINS + 2:_NBINS + 3, :] + nin_sc

    prop = counts / float(nrows)
    safe = jnp.maximum(counts, 1.0)
    contrib = jnp.where(counts > 0.0,
                        jnp.abs(confs / safe - accn / safe) * prop, 0.0)
    sce_ref[...] = jnp.sum(contrib, axis=0, keepdims=True)
    cacc_ref[...] = ncor / nin


def kernel(logits, labels):
    N, C = logits.shape
    nchunks = N // _CHUNK
    c0 = _NT // _CHUNK
    logits_sc = logits.reshape(nchunks, _CHUNK, C)
    labels_sc = labels.reshape(nchunks, _CHUNK)

    mesh = plsc.VectorSubcoreMesh(core_axis_name="c", subcore_axis_name="s")
    sck = functools.partial(
        pl.kernel,
        mesh=mesh,
        compiler_params=pltpu.CompilerParams(needs_layout_passes=False),
        out_type=[
            jax.ShapeDtypeStruct((_NW, _NBINS * 128), jnp.float32),
            jax.ShapeDtypeStruct((_NW, _NBINS * 128), jnp.float32),
            jax.ShapeDtypeStruct((_NW, _NBINS * 128), jnp.float32),
            jax.ShapeDtypeStruct((_NW, 128), jnp.float32),
            jax.ShapeDtypeStruct((_NW, 128), jnp.float32),
        ],
        scratch_types=[
            pltpu.VMEM((_CHUNK, _C), jnp.float32),
            pltpu.VMEM((_CHUNK,), jnp.int32),
            pltpu.VMEM((_NBINS * 128,), jnp.float32),
            pltpu.VMEM((_NBINS * 128,), jnp.float32),
            pltpu.VMEM((_NBINS * 128,), jnp.float32),
            pltpu.VMEM((128,), jnp.float32),
            pltpu.VMEM((128,), jnp.float32),
        ],
    )(functools.partial(_sc_body, c0, nchunks))
    cnt, cnf, acc, nin, ncor = sck(logits_sc, labels_sc)
    cnt = cnt.reshape(_NW, _NBINS, 128)
    cnf = cnf.reshape(_NW, _NBINS, 128)
    acc = acc.reshape(_NW, _NBINS, 128)

    nblocks = _NT // _RT
    bounds = jnp.linspace(0.0, 1.0, _NBINS + 1).reshape(1, _NBINS + 1)
    labels_tc = labels.reshape(N, 1)

    g, s, a = pl.pallas_call(
        functools.partial(_tc_hist_body, nblocks),
        grid=(nblocks,),
        in_specs=[
            pl.BlockSpec((_RT, C), lambda i: (i, 0)),
            pl.BlockSpec((_RT, 1), lambda i: (i, 0)),
            pl.BlockSpec(memory_space=pltpu.SMEM),
            pl.BlockSpec((1, _NBINS + 1), lambda i: (0, 0)),
        ],
        out_specs=[
            pl.BlockSpec((_NBINS * 8, C), lambda i: (0, 0)),
            pl.BlockSpec((_NBINS * 8, C), lambda i: (0, 0)),
            pl.BlockSpec((_NBINS + 3, C), lambda i: (0, 0)),
        ],
        out_shape=[
            jax.ShapeDtypeStruct((_NBINS * 8, C), jnp.float32),
            jax.ShapeDtypeStruct((_NBINS * 8, C), jnp.float32),
            jax.ShapeDtypeStruct((_NBINS + 3, C), jnp.float32),
        ],
    )(logits, labels_tc, bounds, bounds)

    out = pl.pallas_call(
        functools.partial(_fin_body, N),
        out_shape=[
            jax.ShapeDtypeStruct((1, C), jnp.float32),
            jax.ShapeDtypeStruct((1, C), jnp.float32),
        ],
    )(g, s, a, cnt, cnf, acc, nin, ncor)
    return (out[0].reshape(C), out[1].reshape(C))


# speedup vs baseline: 3.4661x; 1.0839x over previous
import functools

import jax
import jax.numpy as jnp
from jax import lax
from jax.experimental import pallas as pl
from jax.experimental.pallas import tpu as pltpu
from jax.experimental.pallas import tpu_sc as plsc

_NBINS = 15
_C = 100
_CHUNK = 160
_LANES = 16
_NW = 32
_NT = 56000
_RT = 2000


def _sc_body(c0, nchunks, logits_ref, labels_ref,
             cnt_out, cnf_out, acc_out, nin_out, ncor_out,
             buf, labbuf, cnt, cnf, acc, nin, ncor):
    wid = lax.axis_index("s") * 2 + lax.axis_index("c")
    iota = lax.iota(jnp.int32, _LANES)
    ones = jnp.ones((_LANES,), jnp.float32)
    zeros = jnp.zeros((_LANES,), jnp.float32)
    lane0 = iota == 0
    mall = iota < _LANES
    m6 = iota >= 12

    def _zero_row(r, carry):
        sl = pl.ds(r * _LANES, _LANES)
        cnt[sl] = zeros
        cnf[sl] = zeros
        acc[sl] = zeros
        return carry
    lax.fori_loop(0, _NBINS * 8, _zero_row, 0)
    for lq in range(8):
        sl = pl.ds(lq * _LANES, _LANES)
        nin[sl] = zeros
        ncor[sl] = zeros

    def _scat_add(ref, idx, x, mask):
        plsc.addupdate_scatter(ref, [idx], x, mask=mask)

    def _shuf(x, k):
        return x.at[iota ^ k].get(mode="promise_in_bounds")

    def _allreduce(x, op):
        for k in (8, 4, 2, 1):
            x = op(x, _shuf(x, k))
        return x

    def _do_row(r, carry):
        v = [buf[r, pl.ds(16 * j, _LANES)] for j in range(6)]
        v6 = buf[r, pl.ds(84, _LANES)]
        mm = v[0]
        for j in range(1, 6):
            mm = jnp.maximum(mm, v[j])
        mm = jnp.maximum(mm, v6)
        mvec = _allreduce(mm, jnp.maximum)
        e = [jnp.exp(vj - mvec) for vj in v]
        e6 = jnp.exp(v6 - mvec)
        ssum = e[0]
        for j in range(1, 6):
            ssum = ssum + e[j]
        ssum = ssum + jnp.where(m6, e6, zeros)
        svec = _allreduce(ssum, jnp.add)
        p = [ej / svec for ej in e]
        p6 = e6 / svec
        pmax = ones / svec
        tmin = jnp.full((_LANES,), 1000, jnp.int32)
        for j in range(6):
            tmin = jnp.minimum(tmin, jnp.where(p[j] == pmax, iota + 16 * j, 1000))
        tmin = jnp.minimum(tmin, jnp.where((p6 == pmax) & m6, iota + 84, 1000))
        choice = _allreduce(tmin, jnp.minimum)

        for j in range(6):
            u = jnp.minimum((p[j] * 15.0).astype(jnp.int32), 14)
            idx = (u << 7) + (iota + 16 * j)
            _scat_add(cnt, idx, ones, mall)
            _scat_add(cnf, idx, p[j], mall)
        u6 = jnp.minimum((p6 * 15.0).astype(jnp.int32), 14)
        idx6 = (u6 << 7) + (iota + 84)
        _scat_add(cnt, idx6, ones, m6)
        _scat_add(cnf, idx6, p6, m6)

        rb = jnp.minimum(r, _CHUNK - _LANES)
        lv = labbuf[pl.ds(rb, _LANES)]
        lab_vec = lv.at[jnp.full((_LANES,), r - rb)].get(
            mode="promise_in_bounds")
        vlab = plsc.load_gather(buf, [jnp.full((_LANES,), r), lab_vec])
        conf_lab = jnp.exp(vlab - mvec) / svec
        ulab = jnp.minimum((conf_lab * 15.0).astype(jnp.int32), 14)
        _scat_add(acc, (ulab << 7) + lab_vec, ones, lane0)
        _scat_add(nin, lab_vec, ones, lane0)
        eq_vec = jnp.where(choice == lab_vec, 1.0, 0.0).astype(jnp.float32)
        _scat_add(ncor, lab_vec, eq_vec, lane0)
        return carry

    def _chunk_iter(j, carry):
        c = c0 + wid + _NW * j

        @pl.when(c < nchunks)
        def _():
            pltpu.sync_copy(logits_ref.at[c], buf)
            pltpu.sync_copy(labels_ref.at[c], labbuf)
            lax.fori_loop(0, _CHUNK, _do_row, 0)
        return carry
    njw = (nchunks - c0 + _NW - 1) // _NW
    lax.fori_loop(0, njw, _chunk_iter, 0)

    pltpu.sync_copy(cnt, cnt_out.at[wid])
    pltpu.sync_copy(cnf, cnf_out.at[wid])
    pltpu.sync_copy(acc, acc_out.at[wid])
    pltpu.sync_copy(nin, nin_out.at[wid])
    pltpu.sync_copy(ncor, ncor_out.at[wid])


def _tc_hist_body(nblocks, logits_ref, labels_ref, bs_ref, bv_ref,
                  g_out, s_out, a_out):
    i = pl.program_id(0)
    R, C = logits_ref.shape
    x = logits_ref[...]
    m = jnp.max(x, axis=1, keepdims=True)
    e = jnp.exp(x - m)
    p = e / jnp.sum(e, axis=1, keepdims=True)

    lab = labels_ref[...]
    iota = lax.broadcasted_iota(jnp.int32, (R, C), 1)
    lab_oh = (iota == lab).astype(jnp.float32)

    pm = jnp.max(p, axis=1, keepdims=True)
    choice = jnp.min(jnp.where(p == pm, iota, C), axis=1, keepdims=True)
    eqf = (choice == lab).astype(jnp.float32)

    conf_lab = jnp.sum(p * lab_oh, axis=1, keepdims=True)
    bv = bv_ref[...]
    ecat = jnp.concatenate(
        [(conf_lab > bv).astype(jnp.float32), eqf, jnp.ones_like(eqf)],
        axis=1)
    a_blk = lax.dot_general(ecat, lab_oh, (((0,), (0,)), ((), ())),
                            preferred_element_type=jnp.float32)

    p3 = p.reshape(R // 8, 8, C)
    glist, slist = [], []
    for k in range(_NBINS):
        t = bs_ref[0, k]
        mk = p3 > t
        glist.append(jnp.sum(mk.astype(jnp.float32), axis=0))
        slist.append(jnp.sum(jnp.where(mk, p3, 0.0), axis=0))
    g_blk = jnp.concatenate(glist, axis=0)
    s_blk = jnp.concatenate(slist, axis=0)

    @pl.when(i == 0)
    def _init():
        g_out[...] = jnp.zeros_like(g_out)
        s_out[...] = jnp.zeros_like(s_out)
        a_out[...] = jnp.zeros_like(a_out)

    g_out[...] += g_blk
    s_out[...] += s_blk
    a_out[...] += a_blk


def _fin_body(nrows, g_ref, s_ref, a_ref,
              cnts_ref, cnfs_ref, accs_ref, nins_ref, ncors_ref,
              sce_ref, cacc_ref):
    gm = jnp.sum(g_ref[...].reshape(_NBINS, 8, _C), axis=1)
    sm = jnp.sum(s_ref[...].reshape(_NBINS, 8, _C), axis=1)
    am = a_ref[...]
    zrow = jnp.zeros((1, _C), jnp.float32)
    cnt_sc = jnp.sum(cnts_ref[...], axis=0)[:, :_C]
    cnf_sc = jnp.sum(cnfs_ref[...], axis=0)[:, :_C]
    acc_sc = jnp.sum(accs_ref[...], axis=0)[:, :_C]
    nin_sc = jnp.sum(nins_ref[...], axis=0, keepdims=True)[:, :_C]
    ncor_sc = jnp.sum(ncors_ref[...], axis=0, keepdims=True)[:, :_C]

    counts = gm - jnp.concatenate([gm[1:, :], zrow], axis=0) + cnt_sc
    confs = sm - jnp.concatenate([sm[1:, :], zrow], axis=0) + cnf_sc
    accn = am[0:_NBINS, :] - am[1:_NBINS + 1, :] + acc_sc
    ncor = am[_NBINS + 1:_NBINS + 2, :] + ncor_sc
    nin = am[_NBINS + 2:_NBINS + 3, :] + nin_sc

    prop = counts / float(nrows)
    safe = jnp.maximum(counts, 1.0)
    contrib = jnp.where(counts > 0.0,
                        jnp.abs(confs / safe - accn / safe) * prop, 0.0)
    sce_ref[...] = jnp.sum(contrib, axis=0, keepdims=True)
    cacc_ref[...] = ncor / nin


def kernel(logits, labels):
    N, C = logits.shape
    nchunks = N // _CHUNK
    c0 = _NT // _CHUNK
    logits_sc = logits.reshape(nchunks, _CHUNK, C)
    labels_sc = labels.reshape(nchunks, _CHUNK)

    mesh = plsc.VectorSubcoreMesh(core_axis_name="c", subcore_axis_name="s")
    sck = functools.partial(
        pl.kernel,
        mesh=mesh,
        compiler_params=pltpu.CompilerParams(needs_layout_passes=False),
        out_type=[
            jax.ShapeDtypeStruct((_NW, _NBINS * 128), jnp.float32),
            jax.ShapeDtypeStruct((_NW, _NBINS * 128), jnp.float32),
            jax.ShapeDtypeStruct((_NW, _NBINS * 128), jnp.float32),
            jax.ShapeDtypeStruct((_NW, 128), jnp.float32),
            jax.ShapeDtypeStruct((_NW, 128), jnp.float32),
        ],
        scratch_types=[
            pltpu.VMEM((_CHUNK, _C), jnp.float32),
            pltpu.VMEM((_CHUNK,), jnp.int32),
            pltpu.VMEM((_NBINS * 128,), jnp.float32),
            pltpu.VMEM((_NBINS * 128,), jnp.float32),
            pltpu.VMEM((_NBINS * 128,), jnp.float32),
            pltpu.VMEM((128,), jnp.float32),
            pltpu.VMEM((128,), jnp.float32),
        ],
    )(functools.partial(_sc_body, c0, nchunks))
    cnt, cnf, acc, nin, ncor = sck(logits_sc, labels_sc)
    cnt = cnt.reshape(_NW, _NBINS, 128)
    cnf = cnf.reshape(_NW, _NBINS, 128)
    acc = acc.reshape(_NW, _NBINS, 128)

    nblocks = _NT // _RT
    bounds = jnp.linspace(0.0, 1.0, _NBINS + 1).reshape(1, _NBINS + 1)
    labels_tc = labels.reshape(N, 1)

    g, s, a = pl.pallas_call(
        functools.partial(_tc_hist_body, nblocks),
        grid=(nblocks,),
        in_specs=[
            pl.BlockSpec((_RT, C), lambda i: (i, 0)),
            pl.BlockSpec((_RT, 1), lambda i: (i, 0)),
            pl.BlockSpec(memory_space=pltpu.SMEM),
            pl.BlockSpec((1, _NBINS + 1), lambda i: (0, 0)),
        ],
        out_specs=[
            pl.BlockSpec((_NBINS * 8, C), lambda i: (0, 0)),
            pl.BlockSpec((_NBINS * 8, C), lambda i: (0, 0)),
            pl.BlockSpec((_NBINS + 3, C), lambda i: (0, 0)),
        ],
        out_shape=[
            jax.ShapeDtypeStruct((_NBINS * 8, C), jnp.float32),
            jax.ShapeDtypeStruct((_NBINS * 8, C), jnp.float32),
            jax.ShapeDtypeStruct((_NBINS + 3, C), jnp.float32),
        ],
    )(logits, labels_tc, bounds, bounds)

    out = pl.pallas_call(
        functools.partial(_fin_body, N),
        out_shape=[
            jax.ShapeDtypeStruct((1, C), jnp.float32),
            jax.ShapeDtypeStruct((1, C), jnp.float32),
        ],
    )(g, s, a, cnt, cnf, acc, nin, ncor)
    return (out[0].reshape(C), out[1].reshape(C))


# speedup vs baseline: 3.6887x; 1.0642x over previous
import functools

import jax
import jax.numpy as jnp
from jax import lax
from jax.experimental import pallas as pl
from jax.experimental.pallas import tpu as pltpu
from jax.experimental.pallas import tpu_sc as plsc

_NBINS = 15
_C = 100
_CHUNK = 160
_LANES = 16
_NW = 32
_NT = 48000
_RT = 2000


def _sc_body(c0, nchunks, logits_ref, labels_ref,
             cnt_out, cnf_out, acc_out, nin_out, ncor_out,
             buf, labbuf, cnt, cnf, acc, nin, ncor):
    wid = lax.axis_index("s") * 2 + lax.axis_index("c")
    iota = lax.iota(jnp.int32, _LANES)
    ones = jnp.ones((_LANES,), jnp.float32)
    zeros = jnp.zeros((_LANES,), jnp.float32)
    lane0 = iota == 0
    mall = iota < _LANES
    m6 = iota >= 12

    def _zero_row(r, carry):
        sl = pl.ds(r * _LANES, _LANES)
        cnt[sl] = zeros
        cnf[sl] = zeros
        acc[sl] = zeros
        return carry
    lax.fori_loop(0, _NBINS * 8, _zero_row, 0)
    for lq in range(8):
        sl = pl.ds(lq * _LANES, _LANES)
        nin[sl] = zeros
        ncor[sl] = zeros

    def _scat_add(ref, idx, x, mask):
        plsc.addupdate_scatter(ref, [idx], x, mask=mask)

    def _shuf(x, k):
        return x.at[iota ^ k].get(mode="promise_in_bounds")

    def _allreduce(x, op):
        for k in (8, 4, 2, 1):
            x = op(x, _shuf(x, k))
        return x

    def _do_row(r, carry):
        v = [buf[r, pl.ds(16 * j, _LANES)] for j in range(6)]
        v6 = buf[r, pl.ds(84, _LANES)]
        mm = v[0]
        for j in range(1, 6):
            mm = jnp.maximum(mm, v[j])
        mm = jnp.maximum(mm, v6)
        mvec = _allreduce(mm, jnp.maximum)
        e = [jnp.exp(vj - mvec) for vj in v]
        e6 = jnp.exp(v6 - mvec)
        ssum = e[0]
        for j in range(1, 6):
            ssum = ssum + e[j]
        ssum = ssum + jnp.where(m6, e6, zeros)
        svec = _allreduce(ssum, jnp.add)
        p = [ej / svec for ej in e]
        p6 = e6 / svec
        pmax = ones / svec
        tmin = jnp.full((_LANES,), 1000, jnp.int32)
        for j in range(6):
            tmin = jnp.minimum(tmin, jnp.where(p[j] == pmax, iota + 16 * j, 1000))
        tmin = jnp.minimum(tmin, jnp.where((p6 == pmax) & m6, iota + 84, 1000))
        choice = _allreduce(tmin, jnp.minimum)

        for j in range(6):
            u = jnp.minimum((p[j] * 15.0).astype(jnp.int32), 14)
            idx = (u << 7) + (iota + 16 * j)
            _scat_add(cnt, idx, ones, mall)
            _scat_add(cnf, idx, p[j], mall)
        u6 = jnp.minimum((p6 * 15.0).astype(jnp.int32), 14)
        idx6 = (u6 << 7) + (iota + 84)
        _scat_add(cnt, idx6, ones, m6)
        _scat_add(cnf, idx6, p6, m6)

        rb = jnp.minimum(r, _CHUNK - _LANES)
        lv = labbuf[pl.ds(rb, _LANES)]
        lab_vec = lv.at[jnp.full((_LANES,), r - rb)].get(
            mode="promise_in_bounds")
        vlab = plsc.load_gather(buf, [jnp.full((_LANES,), r), lab_vec])
        conf_lab = jnp.exp(vlab - mvec) / svec
        ulab = jnp.minimum((conf_lab * 15.0).astype(jnp.int32), 14)
        _scat_add(acc, (ulab << 7) + lab_vec, ones, lane0)
        _scat_add(nin, lab_vec, ones, lane0)
        eq_vec = jnp.where(choice == lab_vec, 1.0, 0.0).astype(jnp.float32)
        _scat_add(ncor, lab_vec, eq_vec, lane0)
        return carry

    def _chunk_iter(j, carry):
        c = c0 + wid + _NW * j

        @pl.when(c < nchunks)
        def _():
            pltpu.sync_copy(logits_ref.at[c], buf)
            pltpu.sync_copy(labels_ref.at[c], labbuf)
            lax.fori_loop(0, _CHUNK, _do_row, 0)
        return carry
    njw = (nchunks - c0 + _NW - 1) // _NW
    lax.fori_loop(0, njw, _chunk_iter, 0)

    pltpu.sync_copy(cnt, cnt_out.at[wid])
    pltpu.sync_copy(cnf, cnf_out.at[wid])
    pltpu.sync_copy(acc, acc_out.at[wid])
    pltpu.sync_copy(nin, nin_out.at[wid])
    pltpu.sync_copy(ncor, ncor_out.at[wid])


def _tc_hist_body(nblocks, logits_ref, labels_ref, bs_ref, bv_ref,
                  g_out, s_out, a_out):
    i = pl.program_id(0)
    R, C = logits_ref.shape
    x = logits_ref[...]
    m = jnp.max(x, axis=1, keepdims=True)
    e = jnp.exp(x - m)
    p = e / jnp.sum(e, axis=1, keepdims=True)

    lab = labels_ref[...]
    iota = lax.broadcasted_iota(jnp.int32, (R, C), 1)
    lab_oh = (iota == lab).astype(jnp.float32)

    pm = jnp.max(p, axis=1, keepdims=True)
    choice = jnp.min(jnp.where(p == pm, iota, C), axis=1, keepdims=True)
    eqf = (choice == lab).astype(jnp.float32)

    conf_lab = jnp.sum(p * lab_oh, axis=1, keepdims=True)
    bv = bv_ref[...]
    ecat = jnp.concatenate(
        [(conf_lab > bv).astype(jnp.float32), eqf, jnp.ones_like(eqf)],
        axis=1)
    a_blk = lax.dot_general(ecat, lab_oh, (((0,), (0,)), ((), ())),
                            preferred_element_type=jnp.float32)

    p3 = p.reshape(R // 8, 8, C)
    glist, slist = [], []
    for k in range(_NBINS):
        t = bs_ref[0, k]
        mk = p3 > t
        glist.append(jnp.sum(mk.astype(jnp.float32), axis=0))
        slist.append(jnp.sum(jnp.where(mk, p3, 0.0), axis=0))
    g_blk = jnp.concatenate(glist, axis=0)
    s_blk = jnp.concatenate(slist, axis=0)

    @pl.when(i == 0)
    def _init():
        g_out[...] = jnp.zeros_like(g_out)
        s_out[...] = jnp.zeros_like(s_out)
        a_out[...] = jnp.zeros_like(a_out)

    g_out[...] += g_blk
    s_out[...] += s_blk
    a_out[...] += a_blk


def _fin_body(nrows, g_ref, s_ref, a_ref,
              cnts_ref, cnfs_ref, accs_ref, nins_ref, ncors_ref,
              sce_ref, cacc_ref):
    gm = jnp.sum(g_ref[...].reshape(_NBINS, 8, _C), axis=1)
    sm = jnp.sum(s_ref[...].reshape(_NBINS, 8, _C), axis=1)
    am = a_ref[...]
    zrow = jnp.zeros((1, _C), jnp.float32)
    cnt_sc = jnp.sum(cnts_ref[...], axis=0)[:, :_C]
    cnf_sc = jnp.sum(cnfs_ref[...], axis=0)[:, :_C]
    acc_sc = jnp.sum(accs_ref[...], axis=0)[:, :_C]
    nin_sc = jnp.sum(nins_ref[...], axis=0, keepdims=True)[:, :_C]
    ncor_sc = jnp.sum(ncors_ref[...], axis=0, keepdims=True)[:, :_C]

    counts = gm - jnp.concatenate([gm[1:, :], zrow], axis=0) + cnt_sc
    confs = sm - jnp.concatenate([sm[1:, :], zrow], axis=0) + cnf_sc
    accn = am[0:_NBINS, :] - am[1:_NBINS + 1, :] + acc_sc
    ncor = am[_NBINS + 1:_NBINS + 2, :] + ncor_sc
    nin = am[_NBINS + 2:_NBINS + 3, :] + nin_sc

    prop = counts / float(nrows)
    safe = jnp.maximum(counts, 1.0)
    contrib = jnp.where(counts > 0.0,
                        jnp.abs(confs / safe - accn / safe) * prop, 0.0)
    sce_ref[...] = jnp.sum(contrib, axis=0, keepdims=True)
    cacc_ref[...] = ncor / nin


def kernel(logits, labels):
    N, C = logits.shape
    nchunks = N // _CHUNK
    c0 = _NT // _CHUNK
    logits_sc = logits.reshape(nchunks, _CHUNK, C)
    labels_sc = labels.reshape(nchunks, _CHUNK)

    mesh = plsc.VectorSubcoreMesh(core_axis_name="c", subcore_axis_name="s")
    sck = functools.partial(
        pl.kernel,
        mesh=mesh,
        compiler_params=pltpu.CompilerParams(needs_layout_passes=False),
        out_type=[
            jax.ShapeDtypeStruct((_NW, _NBINS * 128), jnp.float32),
            jax.ShapeDtypeStruct((_NW, _NBINS * 128), jnp.float32),
            jax.ShapeDtypeStruct((_NW, _NBINS * 128), jnp.float32),
            jax.ShapeDtypeStruct((_NW, 128), jnp.float32),
            jax.ShapeDtypeStruct((_NW, 128), jnp.float32),
        ],
        scratch_types=[
            pltpu.VMEM((_CHUNK, _C), jnp.float32),
            pltpu.VMEM((_CHUNK,), jnp.int32),
            pltpu.VMEM((_NBINS * 128,), jnp.float32),
            pltpu.VMEM((_NBINS * 128,), jnp.float32),
            pltpu.VMEM((_NBINS * 128,), jnp.float32),
            pltpu.VMEM((128,), jnp.float32),
            pltpu.VMEM((128,), jnp.float32),
        ],
    )(functools.partial(_sc_body, c0, nchunks))
    cnt, cnf, acc, nin, ncor = sck(logits_sc, labels_sc)
    cnt = cnt.reshape(_NW, _NBINS, 128)
    cnf = cnf.reshape(_NW, _NBINS, 128)
    acc = acc.reshape(_NW, _NBINS, 128)

    nblocks = _NT // _RT
    bounds = jnp.linspace(0.0, 1.0, _NBINS + 1).reshape(1, _NBINS + 1)
    labels_tc = labels.reshape(N, 1)

    g, s, a = pl.pallas_call(
        functools.partial(_tc_hist_body, nblocks),
        grid=(nblocks,),
        in_specs=[
            pl.BlockSpec((_RT, C), lambda i: (i, 0)),
            pl.BlockSpec((_RT, 1), lambda i: (i, 0)),
            pl.BlockSpec(memory_space=pltpu.SMEM),
            pl.BlockSpec((1, _NBINS + 1), lambda i: (0, 0)),
        ],
        out_specs=[
            pl.BlockSpec((_NBINS * 8, C), lambda i: (0, 0)),
            pl.BlockSpec((_NBINS * 8, C), lambda i: (0, 0)),
            pl.BlockSpec((_NBINS + 3, C), lambda i: (0, 0)),
        ],
        out_shape=[
            jax.ShapeDtypeStruct((_NBINS * 8, C), jnp.float32),
            jax.ShapeDtypeStruct((_NBINS * 8, C), jnp.float32),
            jax.ShapeDtypeStruct((_NBINS + 3, C), jnp.float32),
        ],
    )(logits, labels_tc, bounds, bounds)

    out = pl.pallas_call(
        functools.partial(_fin_body, N),
        out_shape=[
            jax.ShapeDtypeStruct((1, C), jnp.float32),
            jax.ShapeDtypeStruct((1, C), jnp.float32),
        ],
    )(g, s, a, cnt, cnf, acc, nin, ncor)
    return (out[0].reshape(C), out[1].reshape(C))
